# Initial kernel scaffold; baseline (speedup 1.0000x reference)
import jax
import jax.numpy as jnp
from jax.experimental import pallas as pl


def kernel(x, edge_index, W):
    raise NotImplementedError("write your pallas kernel here")


# speedup vs baseline: 4.7874x; 4.7874x over previous
import functools

import jax
import jax.numpy as jnp
from jax import lax
from jax.experimental import pallas as pl
from jax.experimental.pallas import tpu as pltpu
from jax.experimental.pallas import tpu_sc as plsc

NC = 2
NS = 16
NW = NC * NS
LANES = 16
K = 80


def _mesh():
    return plsc.VectorSubcoreMesh(core_axis_name="c", subcore_axis_name="s")


@functools.lru_cache(maxsize=None)
def _build_hist(E, N_PAD):
    EPW = E // NW
    ZH = N_PAD // NS
    f32 = jnp.float32
    sds = jax.ShapeDtypeStruct

    @functools.partial(
        pl.kernel,
        out_type=(sds((N_PAD,), f32),) * 4,
        mesh=_mesh(),
        scratch_types=[
            pltpu.VMEM((K,), jnp.int32),
            pltpu.VMEM((K,), jnp.int32),
            pltpu.VMEM((K,), f32),
            pltpu.VMEM((ZH,), f32),
            pltpu.VMEM_SHARED((N_PAD,), f32),
            pltpu.VMEM_SHARED((N_PAD,), f32),
        ],
    )
    def hist_kernel(src_hbm, dst_hbm, hs0, hd0, hs1, hd1,
                    sidx, didx, ones_v, zb, hist_s, hist_d):
        c = lax.axis_index("c")
        s = lax.axis_index("s")
        wid = s * NC + c

        def fill_z(i, _):
            zb[pl.ds(i * LANES, LANES)] = jnp.zeros((LANES,), f32)
            return 0
        lax.fori_loop(0, ZH // LANES, fill_z, 0)

        def fill_o(i, _):
            ones_v[pl.ds(i * LANES, LANES)] = jnp.ones((LANES,), f32)
            return 0
        lax.fori_loop(0, K // LANES, fill_o, 0)

        r0 = pl.multiple_of(s * ZH, 8)
        pltpu.sync_copy(zb, hist_s.at[pl.ds(r0, ZH)])
        pltpu.sync_copy(zb, hist_d.at[pl.ds(r0, ZH)])
        plsc.subcore_barrier()

        base = wid * EPW

        def chunk(j, _):
            off = pl.multiple_of(base + j * K, 8)
            pltpu.sync_copy(src_hbm.at[pl.ds(off, K)], sidx)
            pltpu.sync_copy(dst_hbm.at[pl.ds(off, K)], didx)
            pltpu.sync_copy(ones_v, hist_s.at[sidx], add=True)
            pltpu.sync_copy(ones_v, hist_d.at[didx], add=True)
            return 0
        lax.fori_loop(0, EPW // K, chunk, 0)
        plsc.subcore_barrier()

        @pl.when(c == 0)
        def _():
            pltpu.sync_copy(hist_s.at[pl.ds(r0, ZH)], hs0.at[pl.ds(r0, ZH)])
            pltpu.sync_copy(hist_d.at[pl.ds(r0, ZH)], hd0.at[pl.ds(r0, ZH)])

        @pl.when(c == 1)
        def _():
            pltpu.sync_copy(hist_s.at[pl.ds(r0, ZH)], hs1.at[pl.ds(r0, ZH)])
            pltpu.sync_copy(hist_d.at[pl.ds(r0, ZH)], hd1.at[pl.ds(r0, ZH)])

    return hist_kernel


@functools.lru_cache(maxsize=None)
def _build_agg(E, N, N_PAD, D):
    EPW = E // NW
    ZR = 80
    RPT = N_PAD // NS
    f32 = jnp.float32
    sds = jax.ShapeDtypeStruct

    @functools.partial(
        pl.kernel,
        out_type=(sds((N_PAD, D), f32), sds((N_PAD, D), f32)),
        mesh=_mesh(),
        scratch_types=[
            pltpu.VMEM((K,), jnp.int32),
            pltpu.VMEM((K,), jnp.int32),
            pltpu.VMEM((K, D), f32),
            pltpu.VMEM((ZR, D), f32),
            pltpu.VMEM_SHARED((N_PAD, D), f32),
            pltpu.SemaphoreType.DMA,
        ],
    )
    def agg_kernel(h_hbm, src_hbm, dst_hbm, agg0, agg1,
                   sidx, didx, rows, zrows, acc, sem):
        c = lax.axis_index("c")
        s = lax.axis_index("s")
        wid = s * NC + c

        def fill_z(r, _):
            for jj in range(D // LANES):
                zrows[r, pl.ds(jj * LANES, LANES)] = jnp.zeros((LANES,), f32)
            return 0
        lax.fori_loop(0, ZR, fill_z, 0)

        base_r = s * RPT
        for b in range(RPT // ZR):
            pltpu.sync_copy(zrows, acc.at[pl.ds(base_r + b * ZR, ZR)])
        plsc.subcore_barrier()

        base = wid * EPW

        def chunk(j, _):
            off = pl.multiple_of(base + j * K, 8)
            pltpu.sync_copy(src_hbm.at[pl.ds(off, K)], sidx)
            pltpu.sync_copy(dst_hbm.at[pl.ds(off, K)], didx)
            pltpu.async_copy(h_hbm.at[sidx], rows, sem).wait()
            pltpu.sync_copy(rows, acc.at[didx], add=True)
            return 0
        lax.fori_loop(0, EPW // K, chunk, 0)
        plsc.subcore_barrier()

        @pl.when(c == 0)
        def _():
            for b in range(RPT // ZR):
                sl = pl.ds(base_r + b * ZR, ZR)
                pltpu.sync_copy(acc.at[sl], agg0.at[sl])

        @pl.when(c == 1)
        def _():
            for b in range(RPT // ZR):
                sl = pl.ds(base_r + b * ZR, ZR)
                pltpu.sync_copy(acc.at[sl], agg1.at[sl])

    return agg_kernel


def _prescale_body(x_ref, a_ref, b_ref, o_ref):
    deg = a_ref[...] + b_ref[...]
    norm = lax.rsqrt(jnp.maximum(deg, 1.0))
    o_ref[...] = x_ref[...] * norm


def _final_body(a0_ref, a1_ref, d0_ref, d1_ref, w_ref, o_ref):
    agg = a0_ref[...] + a1_ref[...]
    deg = d0_ref[...] + d1_ref[...]
    norm = lax.rsqrt(jnp.maximum(deg, 1.0))
    o_ref[...] = jnp.dot(agg * norm, w_ref[...],
                         preferred_element_type=jnp.float32)


def kernel(x, edge_index, W):
    N, D = x.shape
    E = edge_index.shape[1]
    assert E % NW == 0 and (E // NW) % K == 0
    N_PAD = ((N + 639) // 640) * 640
    BR = 400
    assert N % BR == 0
    grid_n = N // BR

    src = edge_index[0]
    dst = edge_index[1]

    hs0, hd0, hs1, hd1 = _build_hist(E, N_PAD)(src, dst)

    h = pl.pallas_call(
        _prescale_body,
        grid=(grid_n,),
        in_specs=[
            pl.BlockSpec((BR, D), lambda i: (i, 0)),
            pl.BlockSpec((BR, 1), lambda i: (i, 0)),
            pl.BlockSpec((BR, 1), lambda i: (i, 0)),
        ],
        out_specs=pl.BlockSpec((BR, D), lambda i: (i, 0)),
        out_shape=jax.ShapeDtypeStruct((N, D), jnp.float32),
    )(x, hs0.reshape(-1, 1), hs1.reshape(-1, 1))

    agg0, agg1 = _build_agg(E, N, N_PAD, D)(h, src, dst)

    out = pl.pallas_call(
        _final_body,
        grid=(grid_n,),
        in_specs=[
            pl.BlockSpec((BR, D), lambda i: (i, 0)),
            pl.BlockSpec((BR, D), lambda i: (i, 0)),
            pl.BlockSpec((BR, 1), lambda i: (i, 0)),
            pl.BlockSpec((BR, 1), lambda i: (i, 0)),
            pl.BlockSpec((D, D), lambda i: (0, 0)),
        ],
        out_specs=pl.BlockSpec((BR, D), lambda i: (i, 0)),
        out_shape=jax.ShapeDtypeStruct((N, D), jnp.float32),
    )(agg0, agg1, hd0.reshape(-1, 1), hd1.reshape(-1, 1), W)

    return out


# speedup vs baseline: 9.9635x; 2.0812x over previous
import functools

import jax
import jax.numpy as jnp
from jax import lax
from jax.experimental import pallas as pl
from jax.experimental.pallas import tpu as pltpu
from jax.experimental.pallas import tpu_sc as plsc

NC = 2
NS = 16
NW = NC * NS
LANES = 16
K = 80


def _mesh():
    return plsc.VectorSubcoreMesh(core_axis_name="c", subcore_axis_name="s")


@functools.lru_cache(maxsize=None)
def _build_hist(E, N_PAD):
    EPW = E // NW
    NCH = EPW // K
    CB = 25
    NB = NCH // CB
    ZH = N_PAD // NS
    FIRE = 5
    f32 = jnp.float32
    sds = jax.ShapeDtypeStruct

    @functools.partial(
        pl.kernel,
        out_type=(sds((N_PAD,), f32),) * 4,
        mesh=_mesh(),
        scratch_types=[
            pltpu.VMEM((NB, CB, K), jnp.int32),
            pltpu.VMEM((NB, CB, K), jnp.int32),
            pltpu.VMEM((K,), f32),
            pltpu.VMEM((ZH,), f32),
            pltpu.VMEM_SHARED((N_PAD,), f32),
            pltpu.VMEM_SHARED((N_PAD,), f32),
            pltpu.SemaphoreType.DMA,
        ],
    )
    def hist_kernel(src_hbm, dst_hbm, hs0, hd0, hs1, hd1,
                    sidx, didx, ones_v, zb, hist_s, hist_d, sem):
        c = lax.axis_index("c")
        s = lax.axis_index("s")
        wid = s * NC + c

        def fill_z(i, _):
            zb[pl.ds(i * LANES, LANES)] = jnp.zeros((LANES,), f32)
            return 0
        lax.fori_loop(0, ZH // LANES, fill_z, 0)

        def fill_o(i, _):
            ones_v[pl.ds(i * LANES, LANES)] = jnp.ones((LANES,), f32)
            return 0
        lax.fori_loop(0, K // LANES, fill_o, 0)

        r0 = pl.multiple_of(s * ZH, 8)
        pltpu.sync_copy(zb, hist_s.at[pl.ds(r0, ZH)])
        pltpu.sync_copy(zb, hist_d.at[pl.ds(r0, ZH)])

        pltpu.sync_copy(src_hbm.at[wid], sidx)
        pltpu.sync_copy(dst_hbm.at[wid], didx)
        plsc.subcore_barrier()

        def fire_block(ob, _):
            for j in range(CB):
                pltpu.async_copy(
                    ones_v, hist_s.at[sidx.at[ob, j]], sem, add=True)
                pltpu.async_copy(
                    ones_v, hist_d.at[didx.at[ob, j]], sem, add=True)
                if j % FIRE == FIRE - 1:
                    for _k in range(2 * FIRE):
                        pltpu.make_async_copy(
                            ones_v, hist_s.at[sidx.at[0, 0]], sem).wait()
            return 0
        lax.fori_loop(0, NB, fire_block, 0)
        plsc.subcore_barrier()

        @pl.when(c == 0)
        def _():
            pltpu.sync_copy(hist_s.at[pl.ds(r0, ZH)], hs0.at[pl.ds(r0, ZH)])
            pltpu.sync_copy(hist_d.at[pl.ds(r0, ZH)], hd0.at[pl.ds(r0, ZH)])

        @pl.when(c == 1)
        def _():
            pltpu.sync_copy(hist_s.at[pl.ds(r0, ZH)], hs1.at[pl.ds(r0, ZH)])
            pltpu.sync_copy(hist_d.at[pl.ds(r0, ZH)], hd1.at[pl.ds(r0, ZH)])

    return hist_kernel


@functools.lru_cache(maxsize=None)
def _build_agg(E, N, N_PAD, D):
    EPW = E // NW
    NCH = EPW // K
    ZR = 80
    RPT = N_PAD // NS
    f32 = jnp.float32
    sds = jax.ShapeDtypeStruct

    CB = 25
    NB = NCH // CB

    @functools.partial(
        pl.kernel,
        out_type=(sds((N_PAD, D), f32), sds((N_PAD, D), f32)),
        mesh=_mesh(),
        scratch_types=[
            pltpu.VMEM((2, CB, K), jnp.int32),
            pltpu.VMEM((2, CB, K), jnp.int32),
            pltpu.VMEM((2, K, D), f32),
            pltpu.VMEM_SHARED((N_PAD, D), f32),
            pltpu.SemaphoreType.DMA,
            pltpu.SemaphoreType.DMA,
            pltpu.SemaphoreType.DMA,
        ],
    )
    def agg_kernel(h_hbm, src_hbm, dst_hbm, agg0, agg1,
                   sidx, didx, rows, acc, semg, sems, semi):
        c = lax.axis_index("c")
        s = lax.axis_index("s")
        wid = s * NC + c

        def fill_z(r, _):
            for jj in range(D // LANES):
                rows[0, r, pl.ds(jj * LANES, LANES)] = jnp.zeros(
                    (LANES,), f32)
            return 0
        lax.fori_loop(0, K, fill_z, 0)

        base_r = s * RPT
        for b in range(RPT // ZR):
            pltpu.async_copy(
                rows.at[0], acc.at[pl.ds(base_r + b * ZR, ZR)], semg)
        for b in range(RPT // ZR):
            pltpu.make_async_copy(
                rows.at[0], acc.at[pl.ds(base_r, ZR)], semg).wait()

        pltpu.sync_copy(src_hbm.at[wid, 0], sidx.at[0])
        pltpu.sync_copy(dst_hbm.at[wid, 0], didx.at[0])
        plsc.subcore_barrier()

        def chunk(cc, _):
            b = lax.rem(cc, 2)
            blk = cc // CB
            j = lax.rem(cc, CB)
            bb = lax.rem(blk, 2)
            jp = lax.rem(cc - 1, CB)
            bp = lax.rem((cc - 1) // CB, 2)

            @pl.when(cc >= 2)
            def _():
                pltpu.make_async_copy(
                    rows.at[0], acc.at[didx.at[0, 0]], sems).wait()

            @pl.when(jnp.logical_and(j == 2, blk + 1 < NB))
            def _():
                pltpu.async_copy(
                    src_hbm.at[wid, blk + 1], sidx.at[1 - bb], semi)
                pltpu.async_copy(
                    dst_hbm.at[wid, blk + 1], didx.at[1 - bb], semi)

            @pl.when(cc >= 1)
            def _():
                pltpu.make_async_copy(
                    h_hbm.at[sidx.at[0, 0]], rows.at[0], semg).wait()

            @pl.when(cc < NCH)
            def _():
                pltpu.async_copy(h_hbm.at[sidx.at[bb, j]], rows.at[b], semg)

            @pl.when(cc >= 1)
            def _():
                pltpu.async_copy(
                    rows.at[1 - b], acc.at[didx.at[bp, jp]], sems, add=True)

            @pl.when(jnp.logical_and(j == CB - 1, blk + 1 < NB))
            def _():
                pltpu.make_async_copy(
                    src_hbm.at[wid, 0], sidx.at[0], semi).wait()
                pltpu.make_async_copy(
                    dst_hbm.at[wid, 0], didx.at[0], semi).wait()
            return 0
        lax.fori_loop(0, NCH + 1, chunk, 0)
        pltpu.make_async_copy(rows.at[0], acc.at[didx.at[0, 0]], sems).wait()
        plsc.subcore_barrier()

        @pl.when(c == 0)
        def _():
            for b in range(RPT // ZR):
                sl = pl.ds(base_r + b * ZR, ZR)
                pltpu.async_copy(acc.at[sl], agg0.at[sl], semg)
            for b in range(RPT // ZR):
                pltpu.make_async_copy(
                    acc.at[pl.ds(base_r, ZR)], agg0.at[pl.ds(base_r, ZR)],
                    semg).wait()

        @pl.when(c == 1)
        def _():
            for b in range(RPT // ZR):
                sl = pl.ds(base_r + b * ZR, ZR)
                pltpu.async_copy(acc.at[sl], agg1.at[sl], semg)
            for b in range(RPT // ZR):
                pltpu.make_async_copy(
                    acc.at[pl.ds(base_r, ZR)], agg1.at[pl.ds(base_r, ZR)],
                    semg).wait()

    return agg_kernel


def _prescale_body(x_ref, a_ref, b_ref, o_ref):
    deg = a_ref[...] + b_ref[...]
    norm = lax.rsqrt(jnp.maximum(deg, 1.0))
    o_ref[...] = x_ref[...] * norm


def _final_body(a0_ref, a1_ref, d0_ref, d1_ref, w_ref, o_ref):
    agg = a0_ref[...] + a1_ref[...]
    deg = d0_ref[...] + d1_ref[...]
    norm = lax.rsqrt(jnp.maximum(deg, 1.0))
    o_ref[...] = jnp.dot(agg * norm, w_ref[...],
                         preferred_element_type=jnp.float32)


def kernel(x, edge_index, W):
    N, D = x.shape
    E = edge_index.shape[1]
    assert E % NW == 0 and (E // NW) % K == 0
    N_PAD = ((N + 639) // 640) * 640
    BR = 400
    assert N % BR == 0
    grid_n = N // BR

    ncht = E // (NW * K)
    src = edge_index[0].reshape(NW, ncht // 25, 25, K)
    dst = edge_index[1].reshape(NW, ncht // 25, 25, K)

    hs0, hd0, hs1, hd1 = _build_hist(E, N_PAD)(src, dst)

    h = pl.pallas_call(
        _prescale_body,
        grid=(grid_n,),
        in_specs=[
            pl.BlockSpec((BR, D), lambda i: (i, 0)),
            pl.BlockSpec((BR, 1), lambda i: (i, 0)),
            pl.BlockSpec((BR, 1), lambda i: (i, 0)),
        ],
        out_specs=pl.BlockSpec((BR, D), lambda i: (i, 0)),
        out_shape=jax.ShapeDtypeStruct((N, D), jnp.float32),
    )(x, hs0.reshape(-1, 1), hs1.reshape(-1, 1))

    agg0, agg1 = _build_agg(E, N, N_PAD, D)(h, src, dst)

    out = pl.pallas_call(
        _final_body,
        grid=(grid_n,),
        in_specs=[
            pl.BlockSpec((BR, D), lambda i: (i, 0)),
            pl.BlockSpec((BR, D), lambda i: (i, 0)),
            pl.BlockSpec((BR, 1), lambda i: (i, 0)),
            pl.BlockSpec((BR, 1), lambda i: (i, 0)),
            pl.BlockSpec((D, D), lambda i: (0, 0)),
        ],
        out_specs=pl.BlockSpec((BR, D), lambda i: (i, 0)),
        out_shape=jax.ShapeDtypeStruct((N, D), jnp.float32),
    )(agg0, agg1, hd0.reshape(-1, 1), hd1.reshape(-1, 1), W)

    return out


# speedup vs baseline: 11.6453x; 1.1688x over previous
import functools

import jax
import jax.numpy as jnp
from jax import lax
from jax.experimental import pallas as pl
from jax.experimental.pallas import tpu as pltpu
from jax.experimental.pallas import tpu_sc as plsc

NC = 2
NS = 16
NW = NC * NS
LANES = 16
K = 80


def _mesh():
    return plsc.VectorSubcoreMesh(core_axis_name="c", subcore_axis_name="s")


@functools.lru_cache(maxsize=None)
def _build_hist(E, N_PAD):
    EPW = E // NW
    NCH = EPW // K
    CB = 25
    NB = NCH // CB
    ZH = N_PAD // NS
    FIRE = 5
    f32 = jnp.float32
    sds = jax.ShapeDtypeStruct

    @functools.partial(
        pl.kernel,
        out_type=(sds((N_PAD,), f32),) * 4,
        mesh=_mesh(),
        scratch_types=[
            pltpu.VMEM((NB, CB, K), jnp.int32),
            pltpu.VMEM((NB, CB, K), jnp.int32),
            pltpu.VMEM((K,), f32),
            pltpu.VMEM((ZH,), f32),
            pltpu.VMEM_SHARED((N_PAD,), f32),
            pltpu.VMEM_SHARED((N_PAD,), f32),
            pltpu.SemaphoreType.DMA,
        ],
    )
    def hist_kernel(src_hbm, dst_hbm, hs0, hd0, hs1, hd1,
                    sidx, didx, ones_v, zb, hist_s, hist_d, sem):
        c = lax.axis_index("c")
        s = lax.axis_index("s")
        wid = s * NC + c

        def fill_z(i, _):
            zb[pl.ds(i * LANES, LANES)] = jnp.zeros((LANES,), f32)
            return 0
        lax.fori_loop(0, ZH // LANES, fill_z, 0)

        def fill_o(i, _):
            ones_v[pl.ds(i * LANES, LANES)] = jnp.ones((LANES,), f32)
            return 0
        lax.fori_loop(0, K // LANES, fill_o, 0)

        r0 = pl.multiple_of(s * ZH, 8)
        pltpu.sync_copy(zb, hist_s.at[pl.ds(r0, ZH)])
        pltpu.sync_copy(zb, hist_d.at[pl.ds(r0, ZH)])

        pltpu.sync_copy(src_hbm.at[wid], sidx)
        pltpu.sync_copy(dst_hbm.at[wid], didx)
        plsc.subcore_barrier()

        def fire_block(ob, _):
            for j in range(CB):
                pltpu.async_copy(
                    ones_v, hist_s.at[sidx.at[ob, j]], sem, add=True)
                pltpu.async_copy(
                    ones_v, hist_d.at[didx.at[ob, j]], sem, add=True)
                if j % FIRE == FIRE - 1:
                    for _k in range(2 * FIRE):
                        pltpu.make_async_copy(
                            ones_v, hist_s.at[sidx.at[0, 0]], sem).wait()
            return 0
        lax.fori_loop(0, NB, fire_block, 0)
        plsc.subcore_barrier()

        @pl.when(c == 0)
        def _():
            pltpu.sync_copy(hist_s.at[pl.ds(r0, ZH)], hs0.at[pl.ds(r0, ZH)])
            pltpu.sync_copy(hist_d.at[pl.ds(r0, ZH)], hd0.at[pl.ds(r0, ZH)])

        @pl.when(c == 1)
        def _():
            pltpu.sync_copy(hist_s.at[pl.ds(r0, ZH)], hs1.at[pl.ds(r0, ZH)])
            pltpu.sync_copy(hist_d.at[pl.ds(r0, ZH)], hd1.at[pl.ds(r0, ZH)])

    return hist_kernel


@functools.lru_cache(maxsize=None)
def _build_agg(E, N, N_PAD, D):
    EPW = E // NW
    NCH = EPW // K
    ZR = 80
    RPT = N_PAD // NS
    f32 = jnp.float32
    sds = jax.ShapeDtypeStruct

    CB = 25
    NB = NCH // CB

    @functools.partial(
        pl.kernel,
        out_type=(sds((N_PAD, D), f32), sds((N_PAD, D), f32)),
        mesh=_mesh(),
        scratch_types=[
            pltpu.VMEM((2, CB, K), jnp.int32),
            pltpu.VMEM((2, CB, K), jnp.int32),
            pltpu.VMEM((2, K, D), f32),
            pltpu.VMEM_SHARED((N_PAD, D), f32),
            pltpu.SemaphoreType.DMA,
            pltpu.SemaphoreType.DMA,
            pltpu.SemaphoreType.DMA,
            pltpu.SemaphoreType.DMA,
        ],
    )
    def agg_kernel(h_hbm, src_hbm, dst_hbm, agg0, agg1,
                   sidx, didx, rows, acc, semg0, semg1, sems, semi):
        c = lax.axis_index("c")
        s = lax.axis_index("s")
        wid = s * NC + c

        def fill_z(r, _):
            for jj in range(D // LANES):
                rows[0, r, pl.ds(jj * LANES, LANES)] = jnp.zeros(
                    (LANES,), f32)
            return 0
        lax.fori_loop(0, K, fill_z, 0)

        base_r = s * RPT
        for b in range(RPT // ZR):
            pltpu.async_copy(
                rows.at[0], acc.at[pl.ds(base_r + b * ZR, ZR)], semg0)
        for b in range(RPT // ZR):
            pltpu.make_async_copy(
                rows.at[0], acc.at[pl.ds(base_r, ZR)], semg0).wait()

        pltpu.sync_copy(src_hbm.at[wid, 0], sidx.at[0])
        pltpu.sync_copy(dst_hbm.at[wid, 0], didx.at[0])
        plsc.subcore_barrier()

        def chunk(cc, _):
            b = lax.rem(cc, 2)
            blk = cc // CB
            j = lax.rem(cc, CB)
            bb = lax.rem(blk, 2)
            jp = lax.rem(cc - 1, CB)
            bp = lax.rem((cc - 1) // CB, 2)

            @pl.when(cc >= 2)
            def _():
                pltpu.make_async_copy(
                    rows.at[0], acc.at[didx.at[0, 0]], sems).wait()

            @pl.when(jnp.logical_and(j == 2, blk + 1 < NB))
            def _():
                pltpu.async_copy(
                    src_hbm.at[wid, blk + 1], sidx.at[1 - bb], semi)
                pltpu.async_copy(
                    dst_hbm.at[wid, blk + 1], didx.at[1 - bb], semi)

            @pl.when(jnp.logical_and(cc < NCH, b == 0))
            def _():
                pltpu.async_copy(h_hbm.at[sidx.at[bb, j]], rows.at[b], semg0)

            @pl.when(jnp.logical_and(cc < NCH, b == 1))
            def _():
                pltpu.async_copy(h_hbm.at[sidx.at[bb, j]], rows.at[b], semg1)

            @pl.when(jnp.logical_and(cc >= 1, b == 1))
            def _():
                pltpu.make_async_copy(
                    h_hbm.at[sidx.at[0, 0]], rows.at[0], semg0).wait()

            @pl.when(jnp.logical_and(cc >= 1, b == 0))
            def _():
                pltpu.make_async_copy(
                    h_hbm.at[sidx.at[0, 0]], rows.at[0], semg1).wait()

            @pl.when(cc >= 1)
            def _():
                pltpu.async_copy(
                    rows.at[1 - b], acc.at[didx.at[bp, jp]], sems, add=True)

            @pl.when(jnp.logical_and(j == CB - 1, blk + 1 < NB))
            def _():
                pltpu.make_async_copy(
                    src_hbm.at[wid, 0], sidx.at[0], semi).wait()
                pltpu.make_async_copy(
                    dst_hbm.at[wid, 0], didx.at[0], semi).wait()

            return 0
        lax.fori_loop(0, NCH + 1, chunk, 0)
        pltpu.make_async_copy(rows.at[0], acc.at[didx.at[0, 0]], sems).wait()
        plsc.subcore_barrier()

        @pl.when(c == 0)
        def _():
            for b in range(RPT // ZR):
                sl = pl.ds(base_r + b * ZR, ZR)
                pltpu.async_copy(acc.at[sl], agg0.at[sl], semg0)
            for b in range(RPT // ZR):
                pltpu.make_async_copy(
                    acc.at[pl.ds(base_r, ZR)], agg0.at[pl.ds(base_r, ZR)],
                    semg0).wait()

        @pl.when(c == 1)
        def _():
            for b in range(RPT // ZR):
                sl = pl.ds(base_r + b * ZR, ZR)
                pltpu.async_copy(acc.at[sl], agg1.at[sl], semg0)
            for b in range(RPT // ZR):
                pltpu.make_async_copy(
                    acc.at[pl.ds(base_r, ZR)], agg1.at[pl.ds(base_r, ZR)],
                    semg0).wait()

    return agg_kernel


def _prescale_body(x_ref, a_ref, b_ref, o_ref):
    deg = a_ref[...] + b_ref[...]
    norm = lax.rsqrt(jnp.maximum(deg, 1.0))
    o_ref[...] = x_ref[...] * norm


def _final_body(a0_ref, a1_ref, d0_ref, d1_ref, w_ref, o_ref):
    agg = a0_ref[...] + a1_ref[...]
    deg = d0_ref[...] + d1_ref[...]
    norm = lax.rsqrt(jnp.maximum(deg, 1.0))
    o_ref[...] = jnp.dot(agg * norm, w_ref[...],
                         preferred_element_type=jnp.float32)


def kernel(x, edge_index, W):
    N, D = x.shape
    E = edge_index.shape[1]
    assert E % NW == 0 and (E // NW) % K == 0
    N_PAD = ((N + 639) // 640) * 640
    BR = 400
    assert N % BR == 0
    grid_n = N // BR

    ncht = E // (NW * K)
    src = edge_index[0].reshape(NW, ncht // 25, 25, K)
    dst = edge_index[1].reshape(NW, ncht // 25, 25, K)

    hs0, hd0, hs1, hd1 = _build_hist(E, N_PAD)(src, dst)

    h = pl.pallas_call(
        _prescale_body,
        grid=(grid_n,),
        in_specs=[
            pl.BlockSpec((BR, D), lambda i: (i, 0)),
            pl.BlockSpec((BR, 1), lambda i: (i, 0)),
            pl.BlockSpec((BR, 1), lambda i: (i, 0)),
        ],
        out_specs=pl.BlockSpec((BR, D), lambda i: (i, 0)),
        out_shape=jax.ShapeDtypeStruct((N, D), jnp.float32),
    )(x, hs0.reshape(-1, 1), hs1.reshape(-1, 1))

    agg0, agg1 = _build_agg(E, N, N_PAD, D)(h, src, dst)

    out = pl.pallas_call(
        _final_body,
        grid=(grid_n,),
        in_specs=[
            pl.BlockSpec((BR, D), lambda i: (i, 0)),
            pl.BlockSpec((BR, D), lambda i: (i, 0)),
            pl.BlockSpec((BR, 1), lambda i: (i, 0)),
            pl.BlockSpec((BR, 1), lambda i: (i, 0)),
            pl.BlockSpec((D, D), lambda i: (0, 0)),
        ],
        out_specs=pl.BlockSpec((BR, D), lambda i: (i, 0)),
        out_shape=jax.ShapeDtypeStruct((N, D), jnp.float32),
    )(agg0, agg1, hd0.reshape(-1, 1), hd1.reshape(-1, 1), W)

    return out
